# baseline (speedup 1.0000x reference)
import functools

import jax
import jax.numpy as jnp
from jax import lax
from jax.experimental import pallas as pl
from jax.experimental.pallas import tpu as pltpu
from jax.experimental.pallas import tpu_sc as plsc

NC = 2
NS = 16
NW = NC * NS
L = 16

MAX_BIN = 256


def _phase1(N, F, CH):
    FG = F // L
    NCHT = N // CH
    SPR = 8
    XR = CH // SPR
    n_iters = -(-NCHT // NW)
    if n_iters % 2:
        n_iters += 1
    HIST = F * MAX_BIN
    NGRP = CH // L

    mesh = plsc.VectorSubcoreMesh(core_axis_name="c", subcore_axis_name="s")

    @functools.partial(
        pl.kernel,
        out_type=jax.ShapeDtypeStruct((NW, 2 * HIST), jnp.float32),
        mesh=mesh,
        compiler_params=pltpu.CompilerParams(needs_layout_passes=False),
        scratch_types=[
            pltpu.VMEM((XR, 8 * F), jnp.int32),
            pltpu.VMEM((XR, 8 * F), jnp.int32),
            pltpu.VMEM((CH,), jnp.float32),
            pltpu.VMEM((CH,), jnp.float32),
            pltpu.VMEM((CH,), jnp.float32),
            pltpu.VMEM((CH,), jnp.float32),
            pltpu.VMEM((HIST,), jnp.float32),
            pltpu.VMEM((HIST,), jnp.float32),
            pltpu.VMEM((HIST,), jnp.float32),
            pltpu.SemaphoreType.DMA,
            pltpu.SemaphoreType.DMA,
        ],
    )
    def k(x_hbm, g_hbm, h_hbm, scr_hbm,
          xb0, xb1, gb0, gb1, hb0, hb1, hg, hh, ht, s0, s1):
        c = lax.axis_index("c")
        s = lax.axis_index("s")
        w = c * NS + s
        sems = (s0, s1)
        xbs, gbs, hbs = (xb0, xb1), (gb0, gb1), (hb0, hb1)

        zeros16 = jnp.zeros((L,), jnp.float32)

        def zero_body(i, _):
            hg[pl.ds(i * L, L)] = zeros16
            hh[pl.ds(i * L, L)] = zeros16
            return 0

        lax.fori_loop(0, HIST // L, zero_body, 0)

        lane = lax.iota(jnp.int32, L)
        bases = [lane + fg * L for fg in range(FG)]

        def copies(ci, b):
            i0 = ci * NW + w
            return (
                pltpu.make_async_copy(x_hbm.at[pl.ds(i0 * XR, XR)], xbs[b],
                                      sems[b]),
                pltpu.make_async_copy(g_hbm.at[pl.ds(i0 * CH, CH)], gbs[b],
                                      sems[b]),
                pltpu.make_async_copy(h_hbm.at[pl.ds(i0 * CH, CH)], hbs[b],
                                      sems[b]),
            )

        def valid(ci):
            return ci * NW + w < NCHT

        def issue(ci, b):
            @pl.when(valid(ci))
            def _():
                for cp in copies(ci, b):
                    cp.start()

        def wait(ci, b):
            for cp in copies(ci, b):
                cp.wait()

        def compute(b):
            @plsc.parallel_loop(0, NGRP, unroll=2)
            def _(gidx):
                row0 = gidx * L
                gvec = gbs[b][pl.ds(row0, L)]
                hvec = hbs[b][pl.ds(row0, L)]
                for i in range(L):
                    gv = jnp.full((L,), gvec[i], jnp.float32)
                    hv = jnp.full((L,), hvec[i], jnp.float32)
                    for fg in range(FG):
                        v = xbs[b][gidx * (L // SPR) + i // SPR,
                                   pl.ds((i % SPR) * F + fg * L, L)]
                        idx = (v << 6) + bases[fg]
                        plsc.addupdate_scatter(hg, [idx], gv)
                        plsc.addupdate_scatter(hh, [idx], hv)

        issue(0, 0)
        issue(1, 1)

        def outer(j, _):
            for b in range(2):
                ci = j * 2 + b

                @pl.when(valid(ci))
                def _():
                    wait(ci, b)
                    compute(b)

                issue(ci + 2, b)
            return 0

        lax.fori_loop(0, n_iters // 2, outer, 0)

        lane_f = lane * F
        for hsrc, off in ((hg, 0), (hh, HIST)):
            def tr_body(f, _):
                for bb in range(MAX_BIN // L):
                    idx = lane_f + (bb * (L * F) + f)
                    ht[pl.ds(f * MAX_BIN + bb * L, L)] = plsc.load_gather(
                        hsrc, [idx])
                return 0

            lax.fori_loop(0, F, tr_body, 0)
            pltpu.sync_copy(ht, scr_hbm.at[w, pl.ds(off, HIST)])

    return k


def _phase2(F):
    HIST = F * MAX_BIN
    ROWS_PER_ARR = F // NW

    mesh = plsc.VectorSubcoreMesh(core_axis_name="c", subcore_axis_name="s")

    @functools.partial(
        pl.kernel,
        out_type=(
            jax.ShapeDtypeStruct((1, F, MAX_BIN), jnp.float32),
            jax.ShapeDtypeStruct((1, F, MAX_BIN), jnp.float32),
        ),
        mesh=mesh,
        compiler_params=pltpu.CompilerParams(needs_layout_passes=False),
        scratch_types=[
            pltpu.VMEM((NW, MAX_BIN), jnp.float32),
            pltpu.VMEM((MAX_BIN,), jnp.float32),
        ],
    )
    def k(scr_hbm, gl_hbm, hl_hbm, acc, row_out):
        c = lax.axis_index("c")
        s = lax.axis_index("s")
        w = c * NS + s

        for a, out_ref in ((0, gl_hbm), (1, hl_hbm)):
            for rr in range(ROWS_PER_ARR):
                f = w * ROWS_PER_ARR + rr
                roff = a * HIST + f * MAX_BIN
                pltpu.sync_copy(scr_hbm.at[:, pl.ds(roff, MAX_BIN)], acc)
                carry = jnp.float32(0.0)
                for kk in range(MAX_BIN // L):
                    v = acc[0, pl.ds(kk * L, L)]
                    for t in range(1, NW):
                        v = v + acc[t, pl.ds(kk * L, L)]
                    pv = plsc.cumsum(v) + jnp.full((L,), carry, jnp.float32)
                    row_out[pl.ds(kk * L, L)] = pv
                    carry = carry + jnp.sum(v)
                pltpu.sync_copy(row_out, out_ref.at[0, f])

    return k


def kernel(X, gradient, hessian):
    N, F = X.shape
    assert F == 64
    CH = 320
    x1 = X.reshape(N // 8, 8 * F)
    scr = _phase1(N, F, CH)(x1, gradient, hessian)
    gl, hl = _phase2(F)(scr)
    return (gl, hl)


# speedup vs baseline: 1.6643x; 1.6643x over previous
import functools

import jax
import jax.numpy as jnp
from jax import lax
from jax.experimental import pallas as pl
from jax.experimental.pallas import tpu as pltpu
from jax.experimental.pallas import tpu_sc as plsc

NC = 2
NS = 16
NW = NC * NS
L = 16

MAX_BIN = 256


def _phase1(N, F, CH):
    FG = F // L
    NCHT = N // CH
    SPW = 4
    XR = CH // SPW
    n_iters = -(-NCHT // NW)
    if n_iters % 2:
        n_iters += 1
    HIST = F * MAX_BIN
    NGRP = CH // L

    mesh = plsc.VectorSubcoreMesh(core_axis_name="c", subcore_axis_name="s")

    @functools.partial(
        pl.kernel,
        out_type=jax.ShapeDtypeStruct((NW, 2 * HIST), jnp.float32),
        mesh=mesh,
        compiler_params=pltpu.CompilerParams(needs_layout_passes=False),
        scratch_types=[
            pltpu.VMEM((XR, F), jnp.int32),
            pltpu.VMEM((XR, F), jnp.int32),
            pltpu.VMEM((CH,), jnp.float32),
            pltpu.VMEM((CH,), jnp.float32),
            pltpu.VMEM((CH,), jnp.float32),
            pltpu.VMEM((CH,), jnp.float32),
            pltpu.VMEM((HIST,), jnp.float32),
            pltpu.VMEM((HIST,), jnp.float32),
            pltpu.VMEM((HIST,), jnp.float32),
            pltpu.SemaphoreType.DMA,
            pltpu.SemaphoreType.DMA,
        ],
    )
    def k(x_hbm, g_hbm, h_hbm, scr_hbm,
          xb0, xb1, gb0, gb1, hb0, hb1, hg, hh, ht, s0, s1):
        c = lax.axis_index("c")
        s = lax.axis_index("s")
        w = c * NS + s
        sems = (s0, s1)
        xbs, gbs, hbs = (xb0, xb1), (gb0, gb1), (hb0, hb1)

        zeros16 = jnp.zeros((L,), jnp.float32)

        def zero_body(i, _):
            hg[pl.ds(i * L, L)] = zeros16
            hh[pl.ds(i * L, L)] = zeros16
            return 0

        lax.fori_loop(0, HIST // L, zero_body, 0)

        lane = lax.iota(jnp.int32, L)
        bases = [lane + fg * L for fg in range(FG)]
        maskb = jnp.full((L,), 0xFF * F, jnp.int32)

        xw_hbm = x_hbm.bitcast(jnp.int32)

        def copies(ci, b):
            i0 = ci * NW + w
            return (
                pltpu.make_async_copy(xw_hbm.at[pl.ds(i0 * XR, XR)], xbs[b],
                                      sems[b]),
                pltpu.make_async_copy(g_hbm.at[pl.ds(i0 * CH, CH)], gbs[b],
                                      sems[b]),
                pltpu.make_async_copy(h_hbm.at[pl.ds(i0 * CH, CH)], hbs[b],
                                      sems[b]),
            )

        def valid(ci):
            return ci * NW + w < NCHT

        def issue(ci, b):
            @pl.when(valid(ci))
            def _():
                for cp in copies(ci, b):
                    cp.start()

        def wait(ci, b):
            for cp in copies(ci, b):
                cp.wait()

        def compute(b):
            @plsc.parallel_loop(0, NGRP, unroll=2)
            def _(gidx):
                row0 = gidx * L
                gvec = gbs[b][pl.ds(row0, L)]
                hvec = hbs[b][pl.ds(row0, L)]
                for r in range(L // SPW):
                    row = gidx * (L // SPW) + r
                    vs = [xbs[b][row, pl.ds(fg * L, L)] for fg in range(FG)]
                    for b2 in range(SPW):
                        i = r * SPW + b2
                        gv = jnp.full((L,), gvec[i], jnp.float32)
                        hv = jnp.full((L,), hvec[i], jnp.float32)
                        sh = 8 * b2 - 6
                        for fg in range(FG):
                            v = vs[fg]
                            bv = ((v >> sh) if sh > 0 else (v << -sh)) \
                                & maskb
                            idx = bv + bases[fg]
                            plsc.addupdate_scatter(hg, [idx], gv)
                            plsc.addupdate_scatter(hh, [idx], hv)

        issue(0, 0)
        issue(1, 1)

        def outer(j, _):
            for b in range(2):
                ci = j * 2 + b

                @pl.when(valid(ci))
                def _():
                    wait(ci, b)
                    compute(b)

                issue(ci + 2, b)
            return 0

        lax.fori_loop(0, n_iters // 2, outer, 0)

        lane_f = lane * F
        for hsrc, off in ((hg, 0), (hh, HIST)):
            def tr_body(f, _):
                for bb in range(MAX_BIN // L):
                    idx = lane_f + (bb * (L * F) + f)
                    ht[pl.ds(f * MAX_BIN + bb * L, L)] = plsc.load_gather(
                        hsrc, [idx])
                return 0

            lax.fori_loop(0, F, tr_body, 0)
            pltpu.sync_copy(ht, scr_hbm.at[w, pl.ds(off, HIST)])

    return k


def _phase2(F):
    HIST = F * MAX_BIN
    ROWS_PER_ARR = F // NW

    mesh = plsc.VectorSubcoreMesh(core_axis_name="c", subcore_axis_name="s")

    @functools.partial(
        pl.kernel,
        out_type=(
            jax.ShapeDtypeStruct((1, F, MAX_BIN), jnp.float32),
            jax.ShapeDtypeStruct((1, F, MAX_BIN), jnp.float32),
        ),
        mesh=mesh,
        compiler_params=pltpu.CompilerParams(needs_layout_passes=False),
        scratch_types=[
            pltpu.VMEM((NW, MAX_BIN), jnp.float32),
            pltpu.VMEM((MAX_BIN,), jnp.float32),
        ],
    )
    def k(scr_hbm, gl_hbm, hl_hbm, acc, row_out):
        c = lax.axis_index("c")
        s = lax.axis_index("s")
        w = c * NS + s

        for a, out_ref in ((0, gl_hbm), (1, hl_hbm)):
            for rr in range(ROWS_PER_ARR):
                f = w * ROWS_PER_ARR + rr
                roff = a * HIST + f * MAX_BIN
                pltpu.sync_copy(scr_hbm.at[:, pl.ds(roff, MAX_BIN)], acc)
                carry = jnp.float32(0.0)
                for kk in range(MAX_BIN // L):
                    v = acc[0, pl.ds(kk * L, L)]
                    for t in range(1, NW):
                        v = v + acc[t, pl.ds(kk * L, L)]
                    pv = plsc.cumsum(v) + jnp.full((L,), carry, jnp.float32)
                    row_out[pl.ds(kk * L, L)] = pv
                    carry = carry + jnp.sum(v)
                pltpu.sync_copy(row_out, out_ref.at[0, f])

    return k


def kernel(X, gradient, hessian):
    N, F = X.shape
    assert F == 64
    CH = 320
    x1 = X.astype(jnp.uint8)
    scr = _phase1(N, F, CH)(x1, gradient, hessian)
    gl, hl = _phase2(F)(scr)
    return (gl, hl)


# speedup vs baseline: 1.6651x; 1.0005x over previous
import functools

import jax
import jax.numpy as jnp
from jax import lax
from jax.experimental import pallas as pl
from jax.experimental.pallas import tpu as pltpu
from jax.experimental.pallas import tpu_sc as plsc

NC = 2
NS = 16
NW = NC * NS
L = 16

MAX_BIN = 256


def _phase1(N, F, CH):
    FG = F // L
    NCHT = N // CH
    SPW = 4
    XR = CH // SPW
    n_iters = -(-NCHT // NW)
    if n_iters % 2:
        n_iters += 1
    HIST = F * MAX_BIN
    NGRP = CH // L

    mesh = plsc.VectorSubcoreMesh(core_axis_name="c", subcore_axis_name="s")

    @functools.partial(
        pl.kernel,
        out_type=jax.ShapeDtypeStruct((NW, 2 * HIST), jnp.float32),
        mesh=mesh,
        compiler_params=pltpu.CompilerParams(
            needs_layout_passes=False, use_tc_tiling_on_sc=True),
        scratch_types=[
            pltpu.VMEM((XR, F), jnp.int32),
            pltpu.VMEM((XR, F), jnp.int32),
            pltpu.VMEM((CH,), jnp.float32),
            pltpu.VMEM((CH,), jnp.float32),
            pltpu.VMEM((CH,), jnp.float32),
            pltpu.VMEM((CH,), jnp.float32),
            pltpu.VMEM((HIST,), jnp.float32),
            pltpu.VMEM((HIST,), jnp.float32),
            pltpu.VMEM((HIST,), jnp.float32),
            pltpu.SemaphoreType.DMA,
            pltpu.SemaphoreType.DMA,
        ],
    )
    def k(x_hbm, g_hbm, h_hbm, scr_hbm,
          xb0, xb1, gb0, gb1, hb0, hb1, hg, hh, ht, s0, s1):
        c = lax.axis_index("c")
        s = lax.axis_index("s")
        w = c * NS + s
        sems = (s0, s1)
        xbs, gbs, hbs = (xb0, xb1), (gb0, gb1), (hb0, hb1)

        zeros16 = jnp.zeros((L,), jnp.float32)

        def zero_body(i, _):
            hg[pl.ds(i * L, L)] = zeros16
            hh[pl.ds(i * L, L)] = zeros16
            return 0

        lax.fori_loop(0, HIST // L, zero_body, 0)

        lane = lax.iota(jnp.int32, L)
        bases = [lane + fg * L for fg in range(FG)]
        maskb = jnp.full((L,), 0xFF * F, jnp.int32)

        xw_hbm = x_hbm.bitcast(jnp.int32)

        def copies(ci, b):
            i0 = ci * NW + w
            return (
                pltpu.make_async_copy(xw_hbm.at[pl.ds(i0 * XR, XR)], xbs[b],
                                      sems[b]),
                pltpu.make_async_copy(g_hbm.at[pl.ds(i0 * CH, CH)], gbs[b],
                                      sems[b]),
                pltpu.make_async_copy(h_hbm.at[pl.ds(i0 * CH, CH)], hbs[b],
                                      sems[b]),
            )

        def valid(ci):
            return ci * NW + w < NCHT

        def issue(ci, b):
            @pl.when(valid(ci))
            def _():
                for cp in copies(ci, b):
                    cp.start()

        def wait(ci, b):
            for cp in copies(ci, b):
                cp.wait()

        def compute(b):
            @plsc.parallel_loop(0, NGRP, unroll=2)
            def _(gidx):
                row0 = gidx * L
                gvec = gbs[b][pl.ds(row0, L)]
                hvec = hbs[b][pl.ds(row0, L)]
                for r in range(L // SPW):
                    row = gidx * (L // SPW) + r
                    vs = [xbs[b][row, pl.ds(fg * L, L)] for fg in range(FG)]
                    for b2 in range(SPW):
                        i = r * SPW + b2
                        gv = jnp.full((L,), gvec[i], jnp.float32)
                        hv = jnp.full((L,), hvec[i], jnp.float32)
                        sh = 8 * b2 - 6
                        for fg in range(FG):
                            v = vs[fg]
                            bv = ((v >> sh) if sh > 0 else (v << -sh)) \
                                & maskb
                            idx = bv + bases[fg]
                            plsc.addupdate_scatter(hg, [idx], gv)
                            plsc.addupdate_scatter(hh, [idx], hv)

        issue(0, 0)
        issue(1, 1)

        def outer(j, _):
            for b in range(2):
                ci = j * 2 + b

                @pl.when(valid(ci))
                def _():
                    wait(ci, b)
                    compute(b)

                issue(ci + 2, b)
            return 0

        lax.fori_loop(0, n_iters // 2, outer, 0)

        lane_f = lane * F
        for hsrc, off in ((hg, 0), (hh, HIST)):
            def tr_body(f, _):
                for bb in range(MAX_BIN // L):
                    idx = lane_f + (bb * (L * F) + f)
                    ht[pl.ds(f * MAX_BIN + bb * L, L)] = plsc.load_gather(
                        hsrc, [idx])
                return 0

            lax.fori_loop(0, F, tr_body, 0)
            pltpu.sync_copy(ht, scr_hbm.at[w, pl.ds(off, HIST)])

    return k


def _phase2(F):
    HIST = F * MAX_BIN
    ROWS_PER_ARR = F // NW

    mesh = plsc.VectorSubcoreMesh(core_axis_name="c", subcore_axis_name="s")

    @functools.partial(
        pl.kernel,
        out_type=(
            jax.ShapeDtypeStruct((1, F, MAX_BIN), jnp.float32),
            jax.ShapeDtypeStruct((1, F, MAX_BIN), jnp.float32),
        ),
        mesh=mesh,
        compiler_params=pltpu.CompilerParams(needs_layout_passes=False),
        scratch_types=[
            pltpu.VMEM((NW, MAX_BIN), jnp.float32),
            pltpu.VMEM((MAX_BIN,), jnp.float32),
        ],
    )
    def k(scr_hbm, gl_hbm, hl_hbm, acc, row_out):
        c = lax.axis_index("c")
        s = lax.axis_index("s")
        w = c * NS + s

        for a, out_ref in ((0, gl_hbm), (1, hl_hbm)):
            for rr in range(ROWS_PER_ARR):
                f = w * ROWS_PER_ARR + rr
                roff = a * HIST + f * MAX_BIN
                pltpu.sync_copy(scr_hbm.at[:, pl.ds(roff, MAX_BIN)], acc)
                carry = jnp.float32(0.0)
                for kk in range(MAX_BIN // L):
                    v = acc[0, pl.ds(kk * L, L)]
                    for t in range(1, NW):
                        v = v + acc[t, pl.ds(kk * L, L)]
                    pv = plsc.cumsum(v) + jnp.full((L,), carry, jnp.float32)
                    row_out[pl.ds(kk * L, L)] = pv
                    carry = carry + jnp.sum(v)
                pltpu.sync_copy(row_out, out_ref.at[0, f])

    return k


def kernel(X, gradient, hessian):
    N, F = X.shape
    assert F == 64
    CH = 320
    x1 = X.astype(jnp.uint8)
    scr = _phase1(N, F, CH)(x1, gradient, hessian)
    gl, hl = _phase2(F)(scr)
    return (gl, hl)


# speedup vs baseline: 1.6675x; 1.0014x over previous
import functools

import jax
import jax.numpy as jnp
from jax import lax
from jax.experimental import pallas as pl
from jax.experimental.pallas import tpu as pltpu
from jax.experimental.pallas import tpu_sc as plsc

NC = 2
NS = 16
NW = NC * NS
L = 16

MAX_BIN = 256


def _phase1(N, F, CH):
    FG = F // L
    NCHT = N // CH
    SPW = 4
    XR = CH // SPW
    n_iters = -(-NCHT // NW)
    if n_iters % 2:
        n_iters += 1
    HIST = F * MAX_BIN
    NGRP = CH // L

    mesh = plsc.VectorSubcoreMesh(core_axis_name="c", subcore_axis_name="s")

    @functools.partial(
        pl.kernel,
        out_type=jax.ShapeDtypeStruct((NW, 2 * HIST), jnp.float32),
        mesh=mesh,
        compiler_params=pltpu.CompilerParams(
            needs_layout_passes=False, use_tc_tiling_on_sc=True),
        scratch_types=[
            pltpu.VMEM((XR, F), jnp.int32),
            pltpu.VMEM((XR, F), jnp.int32),
            pltpu.VMEM((CH,), jnp.float32),
            pltpu.VMEM((CH,), jnp.float32),
            pltpu.VMEM((CH,), jnp.float32),
            pltpu.VMEM((CH,), jnp.float32),
            pltpu.VMEM((HIST,), jnp.float32),
            pltpu.VMEM((HIST,), jnp.float32),
            pltpu.VMEM((HIST,), jnp.float32),
            pltpu.SemaphoreType.DMA,
            pltpu.SemaphoreType.DMA,
        ],
    )
    def k(x_hbm, g_hbm, h_hbm, scr_hbm,
          xb0, xb1, gb0, gb1, hb0, hb1, hg, hh, ht, s0, s1):
        c = lax.axis_index("c")
        s = lax.axis_index("s")
        w = c * NS + s
        sems = (s0, s1)
        xbs, gbs, hbs = (xb0, xb1), (gb0, gb1), (hb0, hb1)

        zeros16 = jnp.zeros((L,), jnp.float32)

        def zero_body(i, _):
            hg[pl.ds(i * L, L)] = zeros16
            hh[pl.ds(i * L, L)] = zeros16
            return 0

        lax.fori_loop(0, HIST // L, zero_body, 0)

        lane = lax.iota(jnp.int32, L)
        bases = [lane + fg * L for fg in range(FG)]
        maskb = jnp.full((L,), 0xFF * F, jnp.int32)

        xw_hbm = x_hbm.bitcast(jnp.int32)

        def copies(ci, b):
            i0 = ci * NW + w
            return (
                pltpu.make_async_copy(xw_hbm.at[pl.ds(i0 * XR, XR)], xbs[b],
                                      sems[b]),
                pltpu.make_async_copy(g_hbm.at[pl.ds(i0 * CH, CH)], gbs[b],
                                      sems[b]),
                pltpu.make_async_copy(h_hbm.at[pl.ds(i0 * CH, CH)], hbs[b],
                                      sems[b]),
            )

        def valid(ci):
            return ci * NW + w < NCHT

        def issue(ci, b):
            @pl.when(valid(ci))
            def _():
                for cp in copies(ci, b):
                    cp.start()

        def wait(ci, b):
            for cp in copies(ci, b):
                cp.wait()

        def compute(b):
            @plsc.parallel_loop(0, NGRP, unroll=4)
            def _(gidx):
                row0 = gidx * L
                gvec = gbs[b][pl.ds(row0, L)]
                hvec = hbs[b][pl.ds(row0, L)]
                for r in range(L // SPW):
                    row = gidx * (L // SPW) + r
                    vs = [xbs[b][row, pl.ds(fg * L, L)] for fg in range(FG)]
                    for b2 in range(SPW):
                        i = r * SPW + b2
                        gv = jnp.full((L,), gvec[i], jnp.float32)
                        hv = jnp.full((L,), hvec[i], jnp.float32)
                        sh = 8 * b2 - 6
                        for fg in range(FG):
                            v = vs[fg]
                            bv = ((v >> sh) if sh > 0 else (v << -sh)) \
                                & maskb
                            idx = bv + bases[fg]
                            plsc.addupdate_scatter(hg, [idx], gv)
                            plsc.addupdate_scatter(hh, [idx], hv)

        issue(0, 0)
        issue(1, 1)

        def outer(j, _):
            for b in range(2):
                ci = j * 2 + b

                @pl.when(valid(ci))
                def _():
                    wait(ci, b)
                    compute(b)

                issue(ci + 2, b)
            return 0

        lax.fori_loop(0, n_iters // 2, outer, 0)

        lane_f = lane * F
        for hsrc, off in ((hg, 0), (hh, HIST)):
            def tr_body(f, _):
                for bb in range(MAX_BIN // L):
                    idx = lane_f + (bb * (L * F) + f)
                    ht[pl.ds(f * MAX_BIN + bb * L, L)] = plsc.load_gather(
                        hsrc, [idx])
                return 0

            lax.fori_loop(0, F, tr_body, 0)
            pltpu.sync_copy(ht, scr_hbm.at[w, pl.ds(off, HIST)])

    return k


def _phase2(F):
    HIST = F * MAX_BIN
    ROWS_PER_ARR = F // NW

    mesh = plsc.VectorSubcoreMesh(core_axis_name="c", subcore_axis_name="s")

    @functools.partial(
        pl.kernel,
        out_type=(
            jax.ShapeDtypeStruct((1, F, MAX_BIN), jnp.float32),
            jax.ShapeDtypeStruct((1, F, MAX_BIN), jnp.float32),
        ),
        mesh=mesh,
        compiler_params=pltpu.CompilerParams(needs_layout_passes=False),
        scratch_types=[
            pltpu.VMEM((NW, MAX_BIN), jnp.float32),
            pltpu.VMEM((MAX_BIN,), jnp.float32),
        ],
    )
    def k(scr_hbm, gl_hbm, hl_hbm, acc, row_out):
        c = lax.axis_index("c")
        s = lax.axis_index("s")
        w = c * NS + s

        for a, out_ref in ((0, gl_hbm), (1, hl_hbm)):
            for rr in range(ROWS_PER_ARR):
                f = w * ROWS_PER_ARR + rr
                roff = a * HIST + f * MAX_BIN
                pltpu.sync_copy(scr_hbm.at[:, pl.ds(roff, MAX_BIN)], acc)
                carry = jnp.float32(0.0)
                for kk in range(MAX_BIN // L):
                    v = acc[0, pl.ds(kk * L, L)]
                    for t in range(1, NW):
                        v = v + acc[t, pl.ds(kk * L, L)]
                    pv = plsc.cumsum(v) + jnp.full((L,), carry, jnp.float32)
                    row_out[pl.ds(kk * L, L)] = pv
                    carry = carry + jnp.sum(v)
                pltpu.sync_copy(row_out, out_ref.at[0, f])

    return k


def kernel(X, gradient, hessian):
    N, F = X.shape
    assert F == 64
    CH = 320
    x1 = X.astype(jnp.uint8)
    scr = _phase1(N, F, CH)(x1, gradient, hessian)
    gl, hl = _phase2(F)(scr)
    return (gl, hl)
